# baseline (speedup 1.0000x reference)
import functools
import math

import numpy as np
import jax
import jax.numpy as jnp
from jax import lax
from jax.experimental import pallas as pl
from jax.experimental.pallas import tpu as pltpu

_STRIDES = (4, 8, 16, 32)
_SIZES = (32, 64, 128, 256)
_RATIOS = (0.5, 1.0, 2.0)
_NA = 3
_PRE = 1000
_POST = 300
_THR = 0.7
_NMS_N = 1024


def _anchor_consts(size):
    ws = np.array([size * np.sqrt(1.0 / r) for r in _RATIOS], dtype=np.float32)
    hs = np.array([size * np.sqrt(r) for r in _RATIOS], dtype=np.float32)
    halfw = (np.float32(0.5) * ws).astype(np.float32)
    halfh = (np.float32(0.5) * hs).astype(np.float32)
    return [float(v) for v in halfw], [float(v) for v in halfh]


def _level_body(xf_ref, w9_ref, bc_ref, wh_ref, bh_ref, out_ref, *, nrows, row0, W, stride, size, img_w, img_h):
    Wp = W + 2
    M = nrows * Wp
    acc = jnp.zeros((M, 256), dtype=jnp.float32)
    for k in range(9):
        kh, kw = divmod(k, 3)
        s = kh * Wp + kw
        acc = acc + jnp.dot(xf_ref[s:s + M, :], w9_ref[k], preferred_element_type=jnp.float32)
    h = jax.nn.relu(acc + bc_ref[0:1, :])
    yt = lax.dot_general(wh_ref[...], h, (((1,), (1,)), ((), ())),
                         preferred_element_type=jnp.float32)
    yt = yt + bh_ref[...]
    t = lax.broadcasted_iota(jnp.int32, (1, M), 1)
    w_idx = (t % Wp).astype(jnp.float32)
    h_idx = (t // Wp + row0).astype(jnp.float32)
    cx = (w_idx + 0.5) * float(stride)
    cy = (h_idx + 0.5) * float(stride)
    halfw, halfh = _anchor_consts(size)
    for a in range(_NA):
        l0 = yt[2 * a:2 * a + 1, :]
        l1 = yt[2 * a + 1:2 * a + 2, :]
        m = jnp.maximum(l0, l1)
        e0 = jnp.exp(l0 - m)
        e1 = jnp.exp(l1 - m)
        score = e1 / (e0 + e1)
        x1a = cx - halfw[a]
        x2a = cx + halfw[a]
        y1a = cy - halfh[a]
        y2a = cy + halfh[a]
        aw = x2a - x1a
        ah = y2a - y1a
        acx = x1a + 0.5 * aw
        acy = y1a + 0.5 * ah
        dx = yt[6 + 4 * a:7 + 4 * a, :]
        dy = yt[7 + 4 * a:8 + 4 * a, :]
        dw = jnp.clip(yt[8 + 4 * a:9 + 4 * a, :], -4.0, 4.0)
        dh = jnp.clip(yt[9 + 4 * a:10 + 4 * a, :], -4.0, 4.0)
        pcx = dx * aw + acx
        pcy = dy * ah + acy
        pw = jnp.exp(dw) * aw
        ph = jnp.exp(dh) * ah
        x1 = jnp.clip(pcx - 0.5 * pw, 0.0, img_w - 1.0)
        y1 = jnp.clip(pcy - 0.5 * ph, 0.0, img_h - 1.0)
        x2 = jnp.clip(pcx + 0.5 * pw, 0.0, img_w - 1.0)
        y2 = jnp.clip(pcy + 0.5 * ph, 0.0, img_h - 1.0)
        out_ref[a:a + 1, :] = score
        out_ref[3 + 4 * a:4 + 4 * a, :] = x1
        out_ref[4 + 4 * a:5 + 4 * a, :] = y1
        out_ref[5 + 4 * a:6 + 4 * a, :] = x2
        out_ref[6 + 4 * a:7 + 4 * a, :] = y2
    out_ref[15:16, :] = jnp.zeros((1, M), dtype=jnp.float32)


def _run_level(f, wc, bc, wcl, bcl, wrg, brg, stride, size, img_w, img_h, nchunks):
    H, W = f.shape[2], f.shape[3]
    Wp = W + 2
    x = f[0].transpose(1, 2, 0)
    xp = jnp.pad(x, ((1, 1), (1, 1), (0, 0)))
    xf = jnp.pad(xp.reshape((H + 2) * Wp, 256), ((0, 2), (0, 0)))
    w9 = wc.transpose(2, 3, 1, 0).reshape(9, 256, 256)
    wh = jnp.concatenate([wcl[:, :, 0, 0], wrg[:, :, 0, 0]], axis=0)
    wh = jnp.pad(wh, ((0, 14), (0, 0)))
    bh = jnp.pad(jnp.concatenate([bcl, brg]), (0, 14))[:, None]
    nrows = H // nchunks
    outs = []
    for c in range(nchunks):
        row0 = c * nrows
        xf_c = xf[row0 * Wp:(row0 + nrows + 2) * Wp + 2]
        body = functools.partial(_level_body, nrows=nrows, row0=row0, W=W,
                                 stride=stride, size=size, img_w=img_w, img_h=img_h)
        out = pl.pallas_call(
            body,
            out_shape=jax.ShapeDtypeStruct((16, nrows * Wp), jnp.float32),
        )(xf_c, w9, bc[None, :], wh, bh)
        outs.append(out.reshape(16, nrows, Wp)[:, :, :W])
    out = jnp.concatenate(outs, axis=1)
    scores = out[:3].transpose(1, 2, 0).reshape(-1)
    boxes = out[3:15].transpose(1, 2, 0).reshape(-1, 4)
    return scores, boxes


def _nms_body(bt_ref, bc_ref, keep_ref, iou_scr):
    n = _NMS_N
    x1r = bt_ref[0:1, :]
    y1r = bt_ref[1:2, :]
    x2r = bt_ref[2:3, :]
    y2r = bt_ref[3:4, :]
    x1c = bc_ref[:, 0:1]
    y1c = bc_ref[:, 1:2]
    x2c = bc_ref[:, 2:3]
    y2c = bc_ref[:, 3:4]
    areas_r = (x2r - x1r) * (y2r - y1r)
    areas_c = (x2c - x1c) * (y2c - y1c)
    xx1 = jnp.maximum(x1c, x1r)
    yy1 = jnp.maximum(y1c, y1r)
    xx2 = jnp.minimum(x2c, x2r)
    yy2 = jnp.minimum(y2c, y2r)
    inter = jnp.maximum(xx2 - xx1, 0.0) * jnp.maximum(yy2 - yy1, 0.0)
    iou = inter / (areas_c + areas_r - inter + 1e-9)
    ri = lax.broadcasted_iota(jnp.int32, (n, n), 0)
    ci = lax.broadcasted_iota(jnp.int32, (n, n), 1)
    sup = ((iou > _THR) & (ci > ri)).astype(jnp.float32)
    B = 128
    nb = n // B
    for b in range(nb):
        iou_scr[b] = sup[:, b * B:(b + 1) * B]
    iota128 = lax.broadcasted_iota(jnp.int32, (1, B), 1)
    masks = [jnp.ones((1, B), dtype=jnp.float32) for _ in range(nb)]
    for b in range(nb):
        col0 = b * B

        def inner(i, mb):
            row = iou_scr[b, pl.ds(col0 + i, 1), :]
            alive = jnp.sum(mb * (iota128 == i).astype(jnp.float32))
            return mb * (1.0 - row * alive)

        mb = lax.fori_loop(0, B, inner, masks[b])
        masks[b] = mb
        for b2 in range(b + 1, nb):
            scross = iou_scr[b2, col0:col0 + B, :]
            supp = jnp.dot(mb, scross, preferred_element_type=jnp.float32)
            masks[b2] = masks[b2] * (supp == 0.0).astype(jnp.float32)
    keep_ref[...] = jnp.concatenate(masks, axis=1)


def _nms_keep(bx):
    bpad = jnp.pad(bx, ((0, _NMS_N - _PRE), (0, 0)))
    bt = jnp.pad(bpad.T, ((0, 4), (0, 0)))
    bc = jnp.pad(bpad, ((0, 0), (0, 4)))
    keep = pl.pallas_call(
        _nms_body,
        out_shape=jax.ShapeDtypeStruct((1, _NMS_N), jnp.float32),
        scratch_shapes=[pltpu.VMEM((_NMS_N // 128, _NMS_N, 128), jnp.float32)],
    )(bt, bc)
    return keep[0, :_PRE]


def kernel(images, feat0, feat1, feat2, feat3, w_conv, b_conv, w_cls, b_cls, w_reg, b_reg):
    img_h, img_w = images.shape[2], images.shape[3]
    feats = [feat0, feat1, feat2, feat3]
    scores_all, boxes_all = [], []
    for l, f in enumerate(feats):
        s, b = _run_level(f, w_conv[l], b_conv[l], w_cls[l], b_cls[l],
                          w_reg[l], b_reg[l], _STRIDES[l], _SIZES[l], img_w, img_h,
                          nchunks=4 if l == 0 else 1)
        scores_all.append(s)
        boxes_all.append(b)
    scores = jnp.concatenate(scores_all, 0)
    boxes = jnp.concatenate(boxes_all, 0)
    sc, idx = lax.top_k(scores, _PRE)
    bx = boxes[idx]
    keep = _nms_keep(bx) > 0.5
    msc = jnp.where(keep, sc, -1e9)
    fsc, fidx = lax.top_k(msc, _POST)
    props = jnp.concatenate([bx[fidx], fsc[:, None]], axis=1)
    return props


# speedup vs baseline: 1.3647x; 1.3647x over previous
import functools
import math

import numpy as np
import jax
import jax.numpy as jnp
from jax import lax
from jax.experimental import pallas as pl
from jax.experimental.pallas import tpu as pltpu

_STRIDES = (4, 8, 16, 32)
_SIZES = (32, 64, 128, 256)
_RATIOS = (0.5, 1.0, 2.0)
_NA = 3
_PRE = 1000
_POST = 300
_THR = 0.7
_NMS_N = 1024


def _anchor_consts(size):
    ws = np.array([size * np.sqrt(1.0 / r) for r in _RATIOS], dtype=np.float32)
    hs = np.array([size * np.sqrt(r) for r in _RATIOS], dtype=np.float32)
    halfw = (np.float32(0.5) * ws).astype(np.float32)
    halfh = (np.float32(0.5) * hs).astype(np.float32)
    return [float(v) for v in halfw], [float(v) for v in halfh]


def _level_body(xf_ref, w9_ref, bc_ref, wh_ref, bh_ref, out_ref, *, nrows, row0, W, stride, size, img_w, img_h):
    Wp = W + 2
    M = nrows * Wp
    acc = jnp.zeros((M, 256), dtype=jnp.float32)
    for k in range(9):
        kh, kw = divmod(k, 3)
        s = kh * Wp + kw
        acc = acc + jnp.dot(xf_ref[s:s + M, :], w9_ref[k], preferred_element_type=jnp.float32)
    h = jax.nn.relu(acc + bc_ref[0:1, :])
    yt = lax.dot_general(wh_ref[...], h, (((1,), (1,)), ((), ())),
                         preferred_element_type=jnp.float32)
    yt = yt + bh_ref[...]
    t = lax.broadcasted_iota(jnp.int32, (1, M), 1)
    w_idx = (t % Wp).astype(jnp.float32)
    h_idx = (t // Wp + row0).astype(jnp.float32)
    cx = (w_idx + 0.5) * float(stride)
    cy = (h_idx + 0.5) * float(stride)
    halfw, halfh = _anchor_consts(size)
    for a in range(_NA):
        l0 = yt[2 * a:2 * a + 1, :]
        l1 = yt[2 * a + 1:2 * a + 2, :]
        m = jnp.maximum(l0, l1)
        e0 = jnp.exp(l0 - m)
        e1 = jnp.exp(l1 - m)
        score = e1 / (e0 + e1)
        x1a = cx - halfw[a]
        x2a = cx + halfw[a]
        y1a = cy - halfh[a]
        y2a = cy + halfh[a]
        aw = x2a - x1a
        ah = y2a - y1a
        acx = x1a + 0.5 * aw
        acy = y1a + 0.5 * ah
        dx = yt[6 + 4 * a:7 + 4 * a, :]
        dy = yt[7 + 4 * a:8 + 4 * a, :]
        dw = jnp.clip(yt[8 + 4 * a:9 + 4 * a, :], -4.0, 4.0)
        dh = jnp.clip(yt[9 + 4 * a:10 + 4 * a, :], -4.0, 4.0)
        pcx = dx * aw + acx
        pcy = dy * ah + acy
        pw = jnp.exp(dw) * aw
        ph = jnp.exp(dh) * ah
        x1 = jnp.clip(pcx - 0.5 * pw, 0.0, img_w - 1.0)
        y1 = jnp.clip(pcy - 0.5 * ph, 0.0, img_h - 1.0)
        x2 = jnp.clip(pcx + 0.5 * pw, 0.0, img_w - 1.0)
        y2 = jnp.clip(pcy + 0.5 * ph, 0.0, img_h - 1.0)
        out_ref[a:a + 1, :] = score
        out_ref[3 + 4 * a:4 + 4 * a, :] = x1
        out_ref[4 + 4 * a:5 + 4 * a, :] = y1
        out_ref[5 + 4 * a:6 + 4 * a, :] = x2
        out_ref[6 + 4 * a:7 + 4 * a, :] = y2
    out_ref[15:16, :] = jnp.zeros((1, M), dtype=jnp.float32)


def _run_level(f, wc, bc, wcl, bcl, wrg, brg, stride, size, img_w, img_h, nchunks):
    H, W = f.shape[2], f.shape[3]
    Wp = W + 2
    x = f[0].transpose(1, 2, 0)
    xp = jnp.pad(x, ((1, 1), (1, 1), (0, 0)))
    xf = jnp.pad(xp.reshape((H + 2) * Wp, 256), ((0, 2), (0, 0)))
    w9 = wc.transpose(2, 3, 1, 0).reshape(9, 256, 256)
    wh = jnp.concatenate([wcl[:, :, 0, 0], wrg[:, :, 0, 0]], axis=0)
    wh = jnp.pad(wh, ((0, 14), (0, 0)))
    bh = jnp.pad(jnp.concatenate([bcl, brg]), (0, 14))[:, None]
    nrows = H // nchunks
    outs = []
    for c in range(nchunks):
        row0 = c * nrows
        xf_c = xf[row0 * Wp:(row0 + nrows + 2) * Wp + 2]
        body = functools.partial(_level_body, nrows=nrows, row0=row0, W=W,
                                 stride=stride, size=size, img_w=img_w, img_h=img_h)
        out = pl.pallas_call(
            body,
            out_shape=jax.ShapeDtypeStruct((16, nrows * Wp), jnp.float32),
        )(xf_c, w9, bc[None, :], wh, bh)
        outs.append(out.reshape(16, nrows, Wp)[:, :, :W])
    out = jnp.concatenate(outs, axis=1)
    scores = out[:3].transpose(1, 2, 0).reshape(-1)
    boxes = out[3:15].transpose(1, 2, 0).reshape(-1, 4)
    return scores, boxes


def _nms_body(bt_ref, bc_ref, keep_ref, iou_scr):
    n = _NMS_N
    x1r = bt_ref[0:1, :]
    y1r = bt_ref[1:2, :]
    x2r = bt_ref[2:3, :]
    y2r = bt_ref[3:4, :]
    x1c = bc_ref[:, 0:1]
    y1c = bc_ref[:, 1:2]
    x2c = bc_ref[:, 2:3]
    y2c = bc_ref[:, 3:4]
    areas_r = (x2r - x1r) * (y2r - y1r)
    areas_c = (x2c - x1c) * (y2c - y1c)
    xx1 = jnp.maximum(x1c, x1r)
    yy1 = jnp.maximum(y1c, y1r)
    xx2 = jnp.minimum(x2c, x2r)
    yy2 = jnp.minimum(y2c, y2r)
    inter = jnp.maximum(xx2 - xx1, 0.0) * jnp.maximum(yy2 - yy1, 0.0)
    iou = inter / (areas_c + areas_r - inter + 1e-9)
    ri = lax.broadcasted_iota(jnp.int32, (n, n), 0)
    ci = lax.broadcasted_iota(jnp.int32, (n, n), 1)
    iou_scr[...] = ((iou > _THR) & (ci > ri)).astype(jnp.float32)

    def w_cond(carry):
        _, changed = carry
        return changed

    def w_body(carry):
        k, _ = carry
        supp = jnp.dot(k, iou_scr[...], preferred_element_type=jnp.float32)
        kn = jnp.where(supp == 0.0, 1.0, 0.0).astype(jnp.float32)
        return kn, jnp.any(kn != k)

    k0 = jnp.ones((1, n), dtype=jnp.float32)
    k, _ = lax.while_loop(w_cond, w_body, (k0, jnp.bool_(True)))
    keep_ref[...] = k


def _nms_keep(bx):
    bpad = jnp.pad(bx, ((0, _NMS_N - _PRE), (0, 0)))
    bt = jnp.pad(bpad.T, ((0, 4), (0, 0)))
    bc = jnp.pad(bpad, ((0, 0), (0, 4)))
    keep = pl.pallas_call(
        _nms_body,
        out_shape=jax.ShapeDtypeStruct((1, _NMS_N), jnp.float32),
        scratch_shapes=[pltpu.VMEM((_NMS_N, _NMS_N), jnp.float32)],
    )(bt, bc)
    return keep[0, :_PRE]


def kernel(images, feat0, feat1, feat2, feat3, w_conv, b_conv, w_cls, b_cls, w_reg, b_reg):
    img_h, img_w = images.shape[2], images.shape[3]
    feats = [feat0, feat1, feat2, feat3]
    scores_all, boxes_all = [], []
    for l, f in enumerate(feats):
        s, b = _run_level(f, w_conv[l], b_conv[l], w_cls[l], b_cls[l],
                          w_reg[l], b_reg[l], _STRIDES[l], _SIZES[l], img_w, img_h,
                          nchunks=4 if l == 0 else 1)
        scores_all.append(s)
        boxes_all.append(b)
    scores = jnp.concatenate(scores_all, 0)
    boxes = jnp.concatenate(boxes_all, 0)
    sc, idx = lax.top_k(scores, _PRE)
    bx = boxes[idx]
    keep = _nms_keep(bx) > 0.5
    msc = jnp.where(keep, sc, -1e9)
    fsc, fidx = lax.top_k(msc, _POST)
    props = jnp.concatenate([bx[fidx], fsc[:, None]], axis=1)
    return props
